# baseline (speedup 1.0000x reference)
import functools

import jax
import jax.numpy as jnp
from jax import lax
from jax.experimental import pallas as pl
from jax.experimental.pallas import tpu as pltpu
from jax.experimental.pallas import tpu_sc as plsc

VOC = 100000
D = 16
K = 32
B = 4096


VB = 4096
GRID = (VOC + VB - 1) // VB


def _tc_body(clo, chi, chlo, chhi, oref):
    def table_argmax(lo, hi):
        def half(ref):
            m = ref[:, 0, :]
            am = jnp.zeros(m.shape, jnp.int32)
            for k in range(1, K):
                v = ref[:, k, :]
                gt = v > m
                am = jnp.where(gt, k, am)
                m = jnp.where(gt, v, m)
            return am

        return jnp.concatenate([half(lo), half(hi)], axis=0)

    oref[...] = table_argmax(clo, chi) + table_argmax(chlo, chhi)


def _tc_scan(ct, cht):
    half_spec = [
        pl.BlockSpec((D // 2, K, VB), lambda i: (0, 0, i)),
        pl.BlockSpec((D // 2, K, VB), lambda i: (1, 0, i)),
    ]
    return pl.pallas_call(
        _tc_body,
        grid=(GRID,),
        in_specs=half_spec + half_spec,
        out_specs=pl.BlockSpec((D, VB), lambda i: (0, i)),
        out_shape=jax.ShapeDtypeStruct((D, VOC), jnp.int32),
    )(ct, ct, cht, cht)


NUM_WORKERS = 32
B_PER_W = B // NUM_WORKERS


@functools.partial(
    pl.kernel,
    out_type=jax.ShapeDtypeStruct((B, D), jnp.int32),
    mesh=plsc.VectorSubcoreMesh(core_axis_name="c", subcore_axis_name="s"),
    compiler_params=pltpu.CompilerParams(
        use_tc_tiling_on_sc=False, needs_layout_passes=False
    ),
    scratch_types=[
        pltpu.VMEM((B_PER_W,), jnp.int32),
        pltpu.VMEM((B_PER_W, D), jnp.int32),
        pltpu.SemaphoreType.DMA,
    ],
)
def _sc_gather(idx_hbm, sum_hbm, out_hbm, idx_v, rows_v, sem):
    c = lax.axis_index("c")
    s = lax.axis_index("s")
    wid = s * 2 + c
    base = pl.multiple_of(wid * B_PER_W, B_PER_W)
    pltpu.sync_copy(idx_hbm.at[pl.ds(base, B_PER_W)], idx_v)
    pltpu.async_copy(sum_hbm.at[idx_v], rows_v, sem).wait()
    pltpu.sync_copy(rows_v, out_hbm.at[pl.ds(base, B_PER_W)])


def kernel(voc_idxs, pai_concept, pai_character):
    idx = voc_idxs.astype(jnp.int32)
    ct = jnp.transpose(pai_concept, (1, 2, 0))
    cht = jnp.transpose(pai_character, (1, 2, 0))
    sum_dv = _tc_scan(ct, cht)
    return _sc_gather(idx, sum_dv.T)
